# Initial kernel scaffold; baseline (speedup 1.0000x reference)
import functools

import jax
import jax.numpy as jnp
from jax import lax
from jax.experimental import pallas as pl
from jax.experimental.pallas import tpu as pltpu
from jax.experimental.pallas import tpu_sc as plsc

N = 10000
D = 128
E = 320000
NC = 2
NS = 16
EPW = E // (NC * NS)
CHUNK = 80
NCHUNK = EPW // CHUNK
RPS = N // NS

_mesh = plsc.VectorSubcoreMesh(core_axis_name="c", subcore_axis_name="s")


@functools.partial(
    pl.kernel,
    mesh=_mesh,
    out_type=jax.ShapeDtypeStruct((NC * N, 1), jnp.float32),
    scratch_types=[
        pltpu.VMEM((CHUNK,), jnp.int32),
        pltpu.VMEM((CHUNK, 1), jnp.float32),
        pltpu.VMEM_SHARED((N, 1), jnp.float32),
    ],
)
def _deg_kernel(dst_hbm, ones_hbm, zeros_hbm, out_hbm, didx, onesv, dacc):
    c = lax.axis_index("c")
    s = lax.axis_index("s")
    pltpu.sync_copy(ones_hbm, onesv)
    pltpu.sync_copy(zeros_hbm, dacc.at[pl.ds(s * RPS, RPS)])
    plsc.subcore_barrier()
    ebase = (c * NS + s) * EPW

    def body(i, carry):
        off = ebase + i * CHUNK
        pltpu.sync_copy(dst_hbm.at[pl.ds(off, CHUNK)], didx)
        pltpu.sync_copy(onesv, dacc.at[didx], add=True)
        return carry

    lax.fori_loop(0, NCHUNK, body, 0)
    plsc.subcore_barrier()
    pltpu.sync_copy(dacc.at[pl.ds(s * RPS, RPS)],
                    out_hbm.at[pl.ds(c * N + s * RPS, RPS)])


@functools.partial(
    pl.kernel,
    mesh=_mesh,
    out_type=jax.ShapeDtypeStruct((NC * N, D), jnp.float32),
    scratch_types=[
        pltpu.VMEM((CHUNK,), jnp.int32),
        pltpu.VMEM((CHUNK,), jnp.int32),
        pltpu.VMEM((CHUNK, D), jnp.float32),
        pltpu.VMEM_SHARED((N, D), jnp.float32),
        pltpu.SemaphoreType.DMA,
    ],
)
def _agg_kernel(h_hbm, src_hbm, dst_hbm, zeros_hbm, out_hbm,
                sidx, didx, rows, acc, sem):
    c = lax.axis_index("c")
    s = lax.axis_index("s")
    pltpu.sync_copy(zeros_hbm, acc.at[pl.ds(s * RPS, RPS)])
    plsc.subcore_barrier()
    ebase = (c * NS + s) * EPW

    def body(i, carry):
        off = ebase + i * CHUNK
        pltpu.sync_copy(src_hbm.at[pl.ds(off, CHUNK)], sidx)
        pltpu.sync_copy(dst_hbm.at[pl.ds(off, CHUNK)], didx)
        pltpu.async_copy(h_hbm.at[sidx], rows, sem).wait()
        pltpu.sync_copy(rows, acc.at[didx], add=True)
        return carry

    lax.fori_loop(0, NCHUNK, body, 0)
    plsc.subcore_barrier()
    pltpu.sync_copy(acc.at[pl.ds(s * RPS, RPS)],
                    out_hbm.at[pl.ds(c * N + s * RPS, RPS)])


BT = 1000


def _tc1_body(x_ref, w_ref, b_ref, degp_ref, h1p_ref, self1_ref, dis_ref):
    deg = degp_ref[0, :] + degp_ref[1, :] + 1.0
    dis = lax.rsqrt(deg)[:, None]
    h = jnp.dot(x_ref[...], w_ref[...], preferred_element_type=jnp.float32)
    h1p_ref[...] = h * dis
    self1_ref[...] = h * (dis * dis) + b_ref[...]
    dis_ref[...] = dis


_tc1 = pl.pallas_call(
    _tc1_body,
    grid=(N // BT,),
    in_specs=[
        pl.BlockSpec((BT, D), lambda i: (i, 0)),
        pl.BlockSpec((D, D), lambda i: (0, 0)),
        pl.BlockSpec((1, D), lambda i: (0, 0)),
        pl.BlockSpec((2, BT), lambda i: (0, i)),
    ],
    out_specs=[
        pl.BlockSpec((BT, D), lambda i: (i, 0)),
        pl.BlockSpec((BT, D), lambda i: (i, 0)),
        pl.BlockSpec((BT, 1), lambda i: (i, 0)),
    ],
    out_shape=[
        jax.ShapeDtypeStruct((N, D), jnp.float32),
        jax.ShapeDtypeStruct((N, D), jnp.float32),
        jax.ShapeDtypeStruct((N, 1), jnp.float32),
    ],
)


def _tc2_body(aggp_ref, self1_ref, dis_ref, w_ref, b_ref, h2p_ref, self2_ref):
    dis = dis_ref[...]
    y1 = jnp.maximum((aggp_ref[0] + aggp_ref[1]) * dis + self1_ref[...], 0.0)
    h2 = jnp.dot(y1, w_ref[...], preferred_element_type=jnp.float32)
    h2p_ref[...] = h2 * dis
    self2_ref[...] = h2 * (dis * dis) + b_ref[...]


_tc2 = pl.pallas_call(
    _tc2_body,
    grid=(N // BT,),
    in_specs=[
        pl.BlockSpec((2, BT, D), lambda i: (0, i, 0)),
        pl.BlockSpec((BT, D), lambda i: (i, 0)),
        pl.BlockSpec((BT, 1), lambda i: (i, 0)),
        pl.BlockSpec((D, D), lambda i: (0, 0)),
        pl.BlockSpec((1, D), lambda i: (0, 0)),
    ],
    out_specs=[
        pl.BlockSpec((BT, D), lambda i: (i, 0)),
        pl.BlockSpec((BT, D), lambda i: (i, 0)),
    ],
    out_shape=[
        jax.ShapeDtypeStruct((N, D), jnp.float32),
        jax.ShapeDtypeStruct((N, D), jnp.float32),
    ],
)


def _tc3_body(aggp_ref, self2_ref, dis_ref, out_ref):
    out_ref[...] = jnp.maximum(
        (aggp_ref[0] + aggp_ref[1]) * dis_ref[...] + self2_ref[...], 0.0)


_tc3 = pl.pallas_call(
    _tc3_body,
    grid=(N // BT,),
    in_specs=[
        pl.BlockSpec((2, BT, D), lambda i: (0, i, 0)),
        pl.BlockSpec((BT, D), lambda i: (i, 0)),
        pl.BlockSpec((BT, 1), lambda i: (i, 0)),
    ],
    out_specs=pl.BlockSpec((BT, D), lambda i: (i, 0)),
    out_shape=jax.ShapeDtypeStruct((N, D), jnp.float32),
)


def kernel(x, edge_index, W1, b1, W2, b2):
    ei = edge_index.astype(jnp.int32)
    src = ei[0]
    dst = ei[1]
    ones_c = jnp.ones((CHUNK, 1), jnp.float32)
    zeros_d = jnp.zeros((RPS, 1), jnp.float32)
    zeros_r = jnp.zeros((RPS, D), jnp.float32)

    degp = _deg_kernel(dst, ones_c, zeros_d).reshape(NC, N)
    h1p, self1, dis = _tc1(x, W1, b1[None, :], degp)
    agg1 = _agg_kernel(h1p, src, dst, zeros_r).reshape(NC, N, D)
    h2p, self2 = _tc2(agg1, self1, dis, W2, b2[None, :])
    agg2 = _agg_kernel(h2p, src, dst, zeros_r).reshape(NC, N, D)
    return _tc3(agg2, self2, dis)


# speedup vs baseline: 12.7883x; 12.7883x over previous
import functools

import jax
import jax.numpy as jnp
from jax import lax
from jax.experimental import pallas as pl
from jax.experimental.pallas import tpu as pltpu
from jax.experimental.pallas import tpu_sc as plsc

N = 10000
D = 128
E = 320000
NC = 2
NS = 16
EPW = E // (NC * NS)
CHUNK = 80
NCHUNK = EPW // CHUNK
S0 = 624
LAST_OFF = (NS - 1) * S0
LAST_LEN = N - LAST_OFF

DW = 16

_mesh = plsc.VectorSubcoreMesh(core_axis_name="c", subcore_axis_name="s")


@functools.partial(
    pl.kernel,
    mesh=_mesh,
    out_type=jax.ShapeDtypeStruct((NC * N, DW), jnp.float32),
    scratch_types=[
        pltpu.VMEM((CHUNK,), jnp.int32),
        pltpu.VMEM((CHUNK, DW), jnp.float32),
        pltpu.VMEM_SHARED((N, DW), jnp.float32),
    ],
)
def _deg_kernel(dst_hbm, ones_hbm, zeros_hbm, out_hbm, didx, onesv, dacc):
    c = lax.axis_index("c")
    s = lax.axis_index("s")
    pltpu.sync_copy(ones_hbm, onesv)

    @pl.when(s < NS - 1)
    def _():
        pltpu.sync_copy(zeros_hbm.at[pl.ds(0, S0)], dacc.at[pl.ds(s * S0, S0)])

    @pl.when(s == NS - 1)
    def _():
        pltpu.sync_copy(zeros_hbm, dacc.at[pl.ds(LAST_OFF, LAST_LEN)])

    plsc.subcore_barrier()
    ebase = (c * NS + s) * EPW

    def body(i, carry):
        off = ebase + i * CHUNK
        pltpu.sync_copy(dst_hbm.at[pl.ds(off, CHUNK)], didx)
        pltpu.sync_copy(onesv, dacc.at[didx], add=True)
        return carry

    lax.fori_loop(0, NCHUNK, body, 0)
    plsc.subcore_barrier()

    @pl.when(s < NS - 1)
    def _():
        pltpu.sync_copy(dacc.at[pl.ds(s * S0, S0)],
                        out_hbm.at[pl.ds(c * N + s * S0, S0)])

    @pl.when(s == NS - 1)
    def _():
        pltpu.sync_copy(dacc.at[pl.ds(LAST_OFF, LAST_LEN)],
                        out_hbm.at[pl.ds(c * N + LAST_OFF, LAST_LEN)])


@functools.partial(
    pl.kernel,
    mesh=_mesh,
    out_type=jax.ShapeDtypeStruct((NC * N, D), jnp.float32),
    scratch_types=[
        pltpu.VMEM((CHUNK,), jnp.int32),
        pltpu.VMEM((CHUNK,), jnp.int32),
        pltpu.VMEM((CHUNK, D), jnp.float32),
        pltpu.VMEM_SHARED((N, D), jnp.float32),
        pltpu.SemaphoreType.DMA,
    ],
)
def _agg_kernel(h_hbm, src_hbm, dst_hbm, zeros_hbm, out_hbm,
                sidx, didx, rows, acc, sem):
    c = lax.axis_index("c")
    s = lax.axis_index("s")
    @pl.when(s < NS - 1)
    def _():
        pltpu.sync_copy(zeros_hbm.at[pl.ds(0, S0)], acc.at[pl.ds(s * S0, S0)])

    @pl.when(s == NS - 1)
    def _():
        pltpu.sync_copy(zeros_hbm, acc.at[pl.ds(LAST_OFF, LAST_LEN)])

    plsc.subcore_barrier()
    ebase = (c * NS + s) * EPW

    def body(i, carry):
        off = ebase + i * CHUNK
        pltpu.sync_copy(src_hbm.at[pl.ds(off, CHUNK)], sidx)
        pltpu.sync_copy(dst_hbm.at[pl.ds(off, CHUNK)], didx)
        pltpu.async_copy(h_hbm.at[sidx], rows, sem).wait()
        pltpu.sync_copy(rows, acc.at[didx], add=True)
        return carry

    lax.fori_loop(0, NCHUNK, body, 0)
    plsc.subcore_barrier()

    @pl.when(s < NS - 1)
    def _():
        pltpu.sync_copy(acc.at[pl.ds(s * S0, S0)],
                        out_hbm.at[pl.ds(c * N + s * S0, S0)])

    @pl.when(s == NS - 1)
    def _():
        pltpu.sync_copy(acc.at[pl.ds(LAST_OFF, LAST_LEN)],
                        out_hbm.at[pl.ds(c * N + LAST_OFF, LAST_LEN)])


BT = 1000


def _tc1_body(x_ref, w_ref, b_ref, degp_ref, h1p_ref, self1_ref, dis_ref):
    deg = degp_ref[0, :, 0:1] + degp_ref[1, :, 0:1] + 1.0
    dis = lax.rsqrt(deg)
    h = jnp.dot(x_ref[...], w_ref[...], preferred_element_type=jnp.float32)
    h1p_ref[...] = h * dis
    self1_ref[...] = h * (dis * dis) + b_ref[...]
    dis_ref[...] = dis


_tc1 = pl.pallas_call(
    _tc1_body,
    grid=(N // BT,),
    in_specs=[
        pl.BlockSpec((BT, D), lambda i: (i, 0)),
        pl.BlockSpec((D, D), lambda i: (0, 0)),
        pl.BlockSpec((1, D), lambda i: (0, 0)),
        pl.BlockSpec((2, BT, DW), lambda i: (0, i, 0)),
    ],
    out_specs=[
        pl.BlockSpec((BT, D), lambda i: (i, 0)),
        pl.BlockSpec((BT, D), lambda i: (i, 0)),
        pl.BlockSpec((BT, 1), lambda i: (i, 0)),
    ],
    out_shape=[
        jax.ShapeDtypeStruct((N, D), jnp.float32),
        jax.ShapeDtypeStruct((N, D), jnp.float32),
        jax.ShapeDtypeStruct((N, 1), jnp.float32),
    ],
)


def _tc2_body(aggp_ref, self1_ref, dis_ref, w_ref, b_ref, h2p_ref, self2_ref):
    dis = dis_ref[...]
    y1 = jnp.maximum((aggp_ref[0] + aggp_ref[1]) * dis + self1_ref[...], 0.0)
    h2 = jnp.dot(y1, w_ref[...], preferred_element_type=jnp.float32)
    h2p_ref[...] = h2 * dis
    self2_ref[...] = h2 * (dis * dis) + b_ref[...]


_tc2 = pl.pallas_call(
    _tc2_body,
    grid=(N // BT,),
    in_specs=[
        pl.BlockSpec((2, BT, D), lambda i: (0, i, 0)),
        pl.BlockSpec((BT, D), lambda i: (i, 0)),
        pl.BlockSpec((BT, 1), lambda i: (i, 0)),
        pl.BlockSpec((D, D), lambda i: (0, 0)),
        pl.BlockSpec((1, D), lambda i: (0, 0)),
    ],
    out_specs=[
        pl.BlockSpec((BT, D), lambda i: (i, 0)),
        pl.BlockSpec((BT, D), lambda i: (i, 0)),
    ],
    out_shape=[
        jax.ShapeDtypeStruct((N, D), jnp.float32),
        jax.ShapeDtypeStruct((N, D), jnp.float32),
    ],
)


def _tc3_body(aggp_ref, self2_ref, dis_ref, out_ref):
    out_ref[...] = jnp.maximum(
        (aggp_ref[0] + aggp_ref[1]) * dis_ref[...] + self2_ref[...], 0.0)


_tc3 = pl.pallas_call(
    _tc3_body,
    grid=(N // BT,),
    in_specs=[
        pl.BlockSpec((2, BT, D), lambda i: (0, i, 0)),
        pl.BlockSpec((BT, D), lambda i: (i, 0)),
        pl.BlockSpec((BT, 1), lambda i: (i, 0)),
    ],
    out_specs=pl.BlockSpec((BT, D), lambda i: (i, 0)),
    out_shape=jax.ShapeDtypeStruct((N, D), jnp.float32),
)


def kernel(x, edge_index, W1, b1, W2, b2):
    ei = edge_index.astype(jnp.int32)
    src = ei[0]
    dst = ei[1]
    ones_c = jnp.ones((CHUNK, DW), jnp.float32)
    zeros_d = jnp.zeros((LAST_LEN, DW), jnp.float32)
    zeros_r = jnp.zeros((LAST_LEN, D), jnp.float32)

    degp = _deg_kernel(dst, ones_c, zeros_d).reshape(NC, N, DW)
    h1p, self1, dis = _tc1(x, W1, b1[None, :], degp)
    agg1 = _agg_kernel(h1p, src, dst, zeros_r).reshape(NC, N, D)
    h2p, self2 = _tc2(agg1, self1, dis, W2, b2[None, :])
    agg2 = _agg_kernel(h2p, src, dst, zeros_r).reshape(NC, N, D)
    return _tc3(agg2, self2, dis)
